# baseline (speedup 1.0000x reference)
import jax
import jax.numpy as jnp
from jax import lax
from jax.experimental import pallas as pl
from jax.experimental.pallas import tpu as pltpu
from jax.experimental.pallas import tpu_sc as plsc

_S = 512
_C = 1024
_D = 768
_NC = 2
_NS = 16
_NW = _NC * _NS
_KPW = _S // _NW


def _body(idx_hbm, kv_hbm, out_ref, myidx, bufk, semm, semk, sems):
    wid = lax.axis_index("s") * _NC + lax.axis_index("c")
    kbase = wid * _KPW

    cp_my = pltpu.make_async_copy(idx_hbm.at[pl.ds(kbase, _KPW)], myidx, semm)
    cp_my.start()
    cp_k = pltpu.make_async_copy(kv_hbm.at[pl.ds(kbase, _KPW)], bufk, semk)
    cp_k.start()
    cp_my.wait()
    cp_k.wait()
    pltpu.async_copy(bufk, out_ref.at[myidx], sems).wait()


_sc_scatter = pl.kernel(
    _body,
    out_type=(),
    mesh=plsc.VectorSubcoreMesh(core_axis_name="c", subcore_axis_name="s"),
    scratch_types=[
        pltpu.VMEM((_KPW,), jnp.int32),
        pltpu.VMEM((_KPW, _D), jnp.float32),
        pltpu.SemaphoreType.DMA,
        pltpu.SemaphoreType.DMA,
        pltpu.SemaphoreType.DMA,
    ],
    compiler_params=pltpu.CompilerParams(needs_layout_passes=False),
)


@jax.jit
def kernel(input_pos, k_val, k_cache):
    idx = input_pos.astype(jnp.int32)
    kv = k_val.reshape(_S, _D)
    kc = k_cache.reshape(_C, _D)
    out_ref = jax.new_ref(kc)
    _sc_scatter(idx, kv, out_ref)
    return out_ref[...].reshape(k_cache.shape)


# speedup vs baseline: 1.0003x; 1.0003x over previous
import jax
import jax.numpy as jnp
from jax import lax
from jax.experimental import pallas as pl
from jax.experimental.pallas import tpu as pltpu
from jax.experimental.pallas import tpu_sc as plsc

_S = 512
_C = 1024
_D = 768
_NC = 2
_NS = 16
_NW = _NC * _NS
_KPW = _S // _NW


_H = _KPW // 2


def _body(idx_hbm, kv_hbm, out_ref, myidxa, myidxb, bufka, bufkb,
          semma, semmb, semka, semkb, sems):
    wid = lax.axis_index("s") * _NC + lax.axis_index("c")
    kbase = wid * _KPW

    cp_ma = pltpu.make_async_copy(idx_hbm.at[pl.ds(kbase, _H)], myidxa, semma)
    cp_ma.start()
    cp_ka = pltpu.make_async_copy(kv_hbm.at[pl.ds(kbase, _H)], bufka, semka)
    cp_ka.start()
    cp_mb = pltpu.make_async_copy(
        idx_hbm.at[pl.ds(kbase + _H, _H)], myidxb, semmb)
    cp_mb.start()
    cp_kb = pltpu.make_async_copy(
        kv_hbm.at[pl.ds(kbase + _H, _H)], bufkb, semkb)
    cp_kb.start()

    cp_ma.wait()
    cp_ka.wait()
    sa = pltpu.make_async_copy(bufka, out_ref.at[myidxa], sems)
    sa.start()
    cp_mb.wait()
    cp_kb.wait()
    sb = pltpu.make_async_copy(bufkb, out_ref.at[myidxb], sems)
    sb.start()
    sa.wait()
    sb.wait()


_sc_scatter = pl.kernel(
    _body,
    out_type=(),
    mesh=plsc.VectorSubcoreMesh(core_axis_name="c", subcore_axis_name="s"),
    scratch_types=[
        pltpu.VMEM((_H,), jnp.int32),
        pltpu.VMEM((_H,), jnp.int32),
        pltpu.VMEM((_H, _D), jnp.float32),
        pltpu.VMEM((_H, _D), jnp.float32),
        pltpu.SemaphoreType.DMA,
        pltpu.SemaphoreType.DMA,
        pltpu.SemaphoreType.DMA,
        pltpu.SemaphoreType.DMA,
        pltpu.SemaphoreType.DMA,
    ],
    compiler_params=pltpu.CompilerParams(needs_layout_passes=False),
)


@jax.jit
def kernel(input_pos, k_val, k_cache):
    idx = input_pos.astype(jnp.int32)
    kv = k_val.reshape(_S, _D)
    kc = k_cache.reshape(_C, _D)
    out_ref = jax.new_ref(kc)
    _sc_scatter(idx, kv, out_ref)
    return out_ref[...].reshape(k_cache.shape)
